# baseline (speedup 1.0000x reference)
import jax
import jax.numpy as jnp
from jax.experimental import pallas as pl
from jax.experimental.pallas import tpu as pltpu

F32 = jnp.float32
BF16 = jnp.bfloat16
C = 128
BM = 400

_A, _B, _C, _D, _N1 = 0, 5, 15, 20, 25
BL = BM
_E, _G, _F, _H, _N2 = 0, 10, 15, 25, 30


def _bdot(a, b):
    return jax.lax.dot_general(a.astype(BF16), b.astype(BF16),
                               (((1,), (0,)), ((), ())),
                               preferred_element_type=F32)


def _bdot_tn(a, b):
    return jax.lax.dot_general(a.astype(BF16), b.astype(BF16),
                               (((0,), (0,)), ((), ())),
                               preferred_element_type=F32)


def _phase_idx(i, start1, start2, hi):
    j = jnp.where(i < start2, i - start1, i - start2)
    return jnp.clip(j, 0, hi)


def _call1_body(xc_ref, inc1_ref, inc2_ref, l0_ref, w0_ref,
                y0_ref, b1_ref, b2_ref,
                b0_s, c1_s, z01_s, z12_s, p01_s, p12_s):
    i = pl.program_id(0)

    @pl.when(i == 0)
    def _():
        z01_s[...] = jnp.zeros_like(z01_s)
        z12_s[...] = jnp.zeros_like(z12_s)

    @pl.when(i < _B)
    def _():
        rows = pl.ds(i * BM, BM)
        a = inc1_ref[...]
        t0 = _bdot(a, xc_ref[2000:6000, :])
        b0_s[rows, :C] = xc_ref[pl.ds(i * BM, BM), :]
        b0_s[rows, C:] = t0.astype(BF16)
        z01_s[...] += _bdot_tn(xc_ref[pl.ds(i * BM, BM), :], a)

    @pl.when(i == _B)
    def _():
        p01_s[...] = z01_s[...].T.astype(BF16)

    @pl.when((i >= _B) & (i < _C))
    def _():
        j = i - _B
        a = inc2_ref[...]
        t21 = _bdot(a, xc_ref[6000:8000, :])
        x1rows = xc_ref[pl.ds(2000 + j * BM, BM), :]
        b1_ref[:, :C] = x1rows
        b1_ref[:, C:2 * C] = p01_s[pl.ds(j * BM, BM), :]
        b1_ref[:, 2 * C:] = t21.astype(BF16)
        z12_s[...] += _bdot_tn(x1rows, a)

    @pl.when(i == _C)
    def _():
        p12_s[...] = z12_s[...].T.astype(BF16)

    @pl.when((i >= _C) & (i < _D))
    def _():
        j = i - _C
        rows = pl.ds(j * BL, BL)
        c1_s[rows, :] = _bdot(l0_ref[...], b0_s[...]).astype(BF16)
        b2_ref[:, :C] = xc_ref[pl.ds(6000 + j * BL, BL), :]
        b2_ref[:, C:] = p12_s[rows, :]

    @pl.when(i >= _D)
    def _():
        rows = pl.ds((i - _D) * BL, BL)
        c2 = _bdot(l0_ref[...], c1_s[...])
        y = (_bdot(b0_s[rows, :], w0_ref[:256, :])
             + _bdot(c1_s[rows, :], w0_ref[256:512, :])
             + _bdot(c2.astype(BF16), w0_ref[512:768, :]))
        y0_ref[...] = jnp.maximum(y, 0.0)


def _call2_body(b1_ref, b2_ref, ld1_ref, lu1_ref, ld2_ref, lu2_ref, w_ref,
                y1_ref, y2_ref,
                d1_s, u1_s, e1_s, f1_s):
    i = pl.program_id(0)

    @pl.when(i < _G)
    def _():
        rows = pl.ds(i * BM, BM)
        d1_s[rows, :] = _bdot(ld1_ref[...], b1_ref[...]).astype(BF16)
        u1_s[rows, :] = _bdot(lu1_ref[...], b1_ref[...]).astype(BF16)

    @pl.when((i >= _F) & (i < _H))
    def _():
        rows = pl.ds((i - _F) * BM, BM)
        d2 = _bdot(ld1_ref[...], d1_s[...])
        u2 = _bdot(lu1_ref[...], u1_s[...])
        y = (_bdot(b1_ref[rows, :], w_ref[:384, :])
             + _bdot(d1_s[rows, :], w_ref[384:768, :])
             + _bdot(d2.astype(BF16), w_ref[768:1152, :])
             + _bdot(u1_s[rows, :], w_ref[1152:1536, :])
             + _bdot(u2.astype(BF16), w_ref[1536:1920, :]))
        y1_ref[...] = jnp.maximum(y, 0.0)

    @pl.when((i >= _G) & (i < _F))
    def _():
        rows = pl.ds((i - _G) * BM, BM)
        e1_s[rows, :] = _bdot(ld2_ref[...], b2_ref[...]).astype(BF16)
        f1_s[rows, :] = _bdot(lu2_ref[...], b2_ref[...]).astype(BF16)

    @pl.when(i >= _H)
    def _():
        rows = pl.ds((i - _H) * BM, BM)
        e2 = _bdot(ld2_ref[...], e1_s[...])
        f2 = _bdot(lu2_ref[...], f1_s[...])
        y = (_bdot(b2_ref[rows, :], w_ref[1920:2176, :])
             + _bdot(e1_s[rows, :], w_ref[2176:2432, :])
             + _bdot(e2.astype(BF16), w_ref[2432:2688, :])
             + _bdot(f1_s[rows, :], w_ref[2688:2944, :])
             + _bdot(f2.astype(BF16), w_ref[2944:3200, :]))
        y2_ref[...] = jnp.maximum(y, 0.0)


def kernel(x_0, x_1, x_2, laplacian_0, laplacian_down_1, laplacian_up_1,
           laplacian_down_2, laplacian_up_2, incidence_1, incidence_2,
           w_0, w_1, w_2):
    xc = jnp.concatenate([x_0, x_1, x_2], axis=0).astype(BF16)

    wt0 = jnp.transpose(w_0, (2, 0, 1)).astype(BF16)
    wt1 = jnp.transpose(w_1, (2, 0, 1)).astype(BF16)
    wt2 = jnp.transpose(w_2, (2, 0, 1)).astype(BF16)
    g = lambda wt, idx: wt[jnp.array(idx)].reshape(len(idx) * C, C)
    w0_all = jnp.concatenate(
        [g(wt0, [0, 3]), g(wt0, [1, 4]), g(wt0, [2, 5])], axis=0)
    w12_all = jnp.concatenate(
        [g(wt1, [0, 5, 10]), g(wt1, [1, 6, 11]), g(wt1, [2, 7, 12]),
         g(wt1, [3, 8, 13]), g(wt1, [4, 9, 14]),
         g(wt2, [0, 5]), g(wt2, [1, 6]), g(wt2, [2, 7]),
         g(wt2, [3, 8]), g(wt2, [4, 9])], axis=0)

    full = lambda arr: pl.BlockSpec(arr.shape, lambda i: (0,) * arr.ndim)

    in_specs1 = [
        full(xc),
        pl.BlockSpec((BM, 4000), lambda i: (jnp.clip(i - _A, 0, 4), 0)),
        pl.BlockSpec((BM, 2000), lambda i: (jnp.clip(i - _B, 0, 9), 0)),
        pl.BlockSpec((BL, 2000), lambda i: (_phase_idx(i, _C, _D, 4), 0)),
        full(w0_all),
    ]
    out_specs1 = [
        pl.BlockSpec((BL, C), lambda i: (jnp.clip(i - _D, 0, 4), 0)),
        pl.BlockSpec((BM, 3 * C), lambda i: (jnp.clip(i - _B, 0, 9), 0)),
        pl.BlockSpec((BL, 2 * C), lambda i: (jnp.clip(i - _C, 0, 4), 0)),
    ]
    out_shape1 = [jax.ShapeDtypeStruct((2000, C), F32),
                  jax.ShapeDtypeStruct((4000, 3 * C), BF16),
                  jax.ShapeDtypeStruct((2000, 2 * C), BF16)]
    scratch1 = [pltpu.VMEM((2000, 2 * C), BF16),
                pltpu.VMEM((2000, 2 * C), BF16),
                pltpu.VMEM((C, 4000), F32),
                pltpu.VMEM((C, 2000), F32),
                pltpu.VMEM((4000, C), BF16),
                pltpu.VMEM((2000, C), BF16)]

    y0, b1, b2 = pl.pallas_call(
        _call1_body,
        grid=(_N1,),
        in_specs=in_specs1,
        out_specs=out_specs1,
        out_shape=out_shape1,
        scratch_shapes=scratch1,
    )(xc, incidence_1, incidence_2, laplacian_0, w0_all)

    in_specs2 = [
        full(b1),
        full(b2),
        pl.BlockSpec((BM, 4000),
                     lambda i: (jnp.where(i < _G, i, jnp.clip(i - _F, 0, 9)), 0)),
        pl.BlockSpec((BM, 4000),
                     lambda i: (jnp.where(i < _G, i, jnp.clip(i - _F, 0, 9)), 0)),
        pl.BlockSpec((BM, 2000),
                     lambda i: (jnp.where(i < _H - 1, jnp.clip(i - _G, 0, 4),
                                          jnp.clip(i - _H, 0, 4)), 0)),
        pl.BlockSpec((BM, 2000),
                     lambda i: (jnp.where(i < _H - 1, jnp.clip(i - _G, 0, 4),
                                          jnp.clip(i - _H, 0, 4)), 0)),
        full(w12_all),
    ]
    out_specs2 = [
        pl.BlockSpec((BM, C), lambda i: (jnp.clip(i - _F, 0, 9), 0)),
        pl.BlockSpec((BM, C), lambda i: (jnp.clip(i - _H, 0, 4), 0)),
    ]
    out_shape2 = [jax.ShapeDtypeStruct((4000, C), F32),
                  jax.ShapeDtypeStruct((2000, C), F32)]
    scratch2 = [pltpu.VMEM((4000, 3 * C), BF16),
                pltpu.VMEM((4000, 3 * C), BF16),
                pltpu.VMEM((2000, 2 * C), BF16),
                pltpu.VMEM((2000, 2 * C), BF16)]

    y1, y2 = pl.pallas_call(
        _call2_body,
        grid=(_N2,),
        in_specs=in_specs2,
        out_specs=out_specs2,
        out_shape=out_shape2,
        scratch_shapes=scratch2,
    )(b1, b2, laplacian_down_1, laplacian_up_1,
      laplacian_down_2, laplacian_up_2, w12_all)
    return (y0, y1, y2)


# speedup vs baseline: 1.0098x; 1.0098x over previous
import jax
import jax.numpy as jnp
from jax.experimental import pallas as pl
from jax.experimental.pallas import tpu as pltpu

F32 = jnp.float32
BF16 = jnp.bfloat16
C = 128
BM = 400

_A, _B, _C, _D, _N1 = 0, 5, 15, 17, 19
BL = 1000
_E, _G, _F, _H, _N2 = 0, 10, 15, 25, 30


def _bdot(a, b):
    return jax.lax.dot_general(a.astype(BF16), b.astype(BF16),
                               (((1,), (0,)), ((), ())),
                               preferred_element_type=F32)


def _bdot_tn(a, b):
    return jax.lax.dot_general(a.astype(BF16), b.astype(BF16),
                               (((0,), (0,)), ((), ())),
                               preferred_element_type=F32)


def _phase_idx(i, start1, start2, hi):
    j = jnp.where(i < start2, i - start1, i - start2)
    return jnp.clip(j, 0, hi)


def _call1_body(xc_ref, inc1_ref, inc2_ref, l0_ref, w0_ref,
                y0_ref, b1_ref, b2_ref,
                b0_s, c1_s, z01_s, z12_s, p01_s, p12_s):
    i = pl.program_id(0)

    @pl.when(i == 0)
    def _():
        z01_s[...] = jnp.zeros_like(z01_s)
        z12_s[...] = jnp.zeros_like(z12_s)

    @pl.when(i < _B)
    def _():
        rows = pl.ds(i * BM, BM)
        a = inc1_ref[...]
        t0 = _bdot(a, xc_ref[2000:6000, :])
        b0_s[rows, :C] = xc_ref[pl.ds(i * BM, BM), :]
        b0_s[rows, C:] = t0.astype(BF16)
        z01_s[...] += _bdot_tn(xc_ref[pl.ds(i * BM, BM), :], a)

    @pl.when(i == _B)
    def _():
        p01_s[...] = z01_s[...].T.astype(BF16)

    @pl.when((i >= _B) & (i < _C))
    def _():
        j = i - _B
        a = inc2_ref[...]
        t21 = _bdot(a, xc_ref[6000:8000, :])
        x1rows = xc_ref[pl.ds(2000 + j * BM, BM), :]
        b1_ref[:, :C] = x1rows
        b1_ref[:, C:2 * C] = p01_s[pl.ds(j * BM, BM), :]
        b1_ref[:, 2 * C:] = t21.astype(BF16)
        z12_s[...] += _bdot_tn(x1rows, a)

    @pl.when(i == _C)
    def _():
        p12_s[...] = z12_s[...].T.astype(BF16)

    @pl.when((i >= _C) & (i < _D))
    def _():
        j = i - _C
        rows = pl.ds(j * BL, BL)
        c1_s[rows, :] = _bdot(l0_ref[...], b0_s[...]).astype(BF16)
        b2_ref[:, :C] = xc_ref[pl.ds(6000 + j * BL, BL), :]
        b2_ref[:, C:] = p12_s[rows, :]

    @pl.when(i >= _D)
    def _():
        rows = pl.ds((i - _D) * BL, BL)
        c2 = _bdot(l0_ref[...], c1_s[...])
        y = (_bdot(b0_s[rows, :], w0_ref[:256, :])
             + _bdot(c1_s[rows, :], w0_ref[256:512, :])
             + _bdot(c2.astype(BF16), w0_ref[512:768, :]))
        y0_ref[...] = jnp.maximum(y, 0.0)


def _call2_body(b1_ref, b2_ref, ld1_ref, lu1_ref, ld2_ref, lu2_ref, w_ref,
                y1_ref, y2_ref,
                d1_s, u1_s, e1_s, f1_s):
    i = pl.program_id(0)

    @pl.when(i < _G)
    def _():
        rows = pl.ds(i * BM, BM)
        d1_s[rows, :] = _bdot(ld1_ref[...], b1_ref[...]).astype(BF16)
        u1_s[rows, :] = _bdot(lu1_ref[...], b1_ref[...]).astype(BF16)

    @pl.when((i >= _F) & (i < _H))
    def _():
        rows = pl.ds((i - _F) * BM, BM)
        d2 = _bdot(ld1_ref[...], d1_s[...])
        u2 = _bdot(lu1_ref[...], u1_s[...])
        y = (_bdot(b1_ref[rows, :], w_ref[:384, :])
             + _bdot(d1_s[rows, :], w_ref[384:768, :])
             + _bdot(d2.astype(BF16), w_ref[768:1152, :])
             + _bdot(u1_s[rows, :], w_ref[1152:1536, :])
             + _bdot(u2.astype(BF16), w_ref[1536:1920, :]))
        y1_ref[...] = jnp.maximum(y, 0.0)

    @pl.when((i >= _G) & (i < _F))
    def _():
        rows = pl.ds((i - _G) * BM, BM)
        e1_s[rows, :] = _bdot(ld2_ref[...], b2_ref[...]).astype(BF16)
        f1_s[rows, :] = _bdot(lu2_ref[...], b2_ref[...]).astype(BF16)

    @pl.when(i >= _H)
    def _():
        rows = pl.ds((i - _H) * BM, BM)
        e2 = _bdot(ld2_ref[...], e1_s[...])
        f2 = _bdot(lu2_ref[...], f1_s[...])
        y = (_bdot(b2_ref[rows, :], w_ref[1920:2176, :])
             + _bdot(e1_s[rows, :], w_ref[2176:2432, :])
             + _bdot(e2.astype(BF16), w_ref[2432:2688, :])
             + _bdot(f1_s[rows, :], w_ref[2688:2944, :])
             + _bdot(f2.astype(BF16), w_ref[2944:3200, :]))
        y2_ref[...] = jnp.maximum(y, 0.0)


def kernel(x_0, x_1, x_2, laplacian_0, laplacian_down_1, laplacian_up_1,
           laplacian_down_2, laplacian_up_2, incidence_1, incidence_2,
           w_0, w_1, w_2):
    xc = jnp.concatenate([x_0, x_1, x_2], axis=0).astype(BF16)

    wt0 = jnp.transpose(w_0, (2, 0, 1)).astype(BF16)
    wt1 = jnp.transpose(w_1, (2, 0, 1)).astype(BF16)
    wt2 = jnp.transpose(w_2, (2, 0, 1)).astype(BF16)
    g = lambda wt, idx: wt[jnp.array(idx)].reshape(len(idx) * C, C)
    w0_all = jnp.concatenate(
        [g(wt0, [0, 3]), g(wt0, [1, 4]), g(wt0, [2, 5])], axis=0)
    w12_all = jnp.concatenate(
        [g(wt1, [0, 5, 10]), g(wt1, [1, 6, 11]), g(wt1, [2, 7, 12]),
         g(wt1, [3, 8, 13]), g(wt1, [4, 9, 14]),
         g(wt2, [0, 5]), g(wt2, [1, 6]), g(wt2, [2, 7]),
         g(wt2, [3, 8]), g(wt2, [4, 9])], axis=0)

    full = lambda arr: pl.BlockSpec(arr.shape, lambda i: (0,) * arr.ndim)

    in_specs1 = [
        full(xc),
        pl.BlockSpec((BM, 4000), lambda i: (jnp.clip(i - _A, 0, 4), 0)),
        pl.BlockSpec((BM, 2000), lambda i: (jnp.clip(i - _B, 0, 9), 0)),
        pl.BlockSpec((BL, 2000), lambda i: (_phase_idx(i, _C, _D, 1), 0)),
        full(w0_all),
    ]
    out_specs1 = [
        pl.BlockSpec((BL, C), lambda i: (jnp.clip(i - _D, 0, 1), 0)),
        pl.BlockSpec((BM, 3 * C), lambda i: (jnp.clip(i - _B, 0, 9), 0)),
        pl.BlockSpec((BL, 2 * C), lambda i: (jnp.clip(i - _C, 0, 1), 0)),
    ]
    out_shape1 = [jax.ShapeDtypeStruct((2000, C), F32),
                  jax.ShapeDtypeStruct((4000, 3 * C), BF16),
                  jax.ShapeDtypeStruct((2000, 2 * C), BF16)]
    scratch1 = [pltpu.VMEM((2000, 2 * C), BF16),
                pltpu.VMEM((2000, 2 * C), BF16),
                pltpu.VMEM((C, 4000), F32),
                pltpu.VMEM((C, 2000), F32),
                pltpu.VMEM((4000, C), BF16),
                pltpu.VMEM((2000, C), BF16)]

    y0, b1, b2 = pl.pallas_call(
        _call1_body,
        grid=(_N1,),
        in_specs=in_specs1,
        out_specs=out_specs1,
        out_shape=out_shape1,
        scratch_shapes=scratch1,
    )(xc, incidence_1, incidence_2, laplacian_0, w0_all)

    in_specs2 = [
        full(b1),
        full(b2),
        pl.BlockSpec((BM, 4000),
                     lambda i: (jnp.where(i < _G, i, jnp.clip(i - _F, 0, 9)), 0)),
        pl.BlockSpec((BM, 4000),
                     lambda i: (jnp.where(i < _G, i, jnp.clip(i - _F, 0, 9)), 0)),
        pl.BlockSpec((BM, 2000),
                     lambda i: (jnp.where(i < _H - 1, jnp.clip(i - _G, 0, 4),
                                          jnp.clip(i - _H, 0, 4)), 0)),
        pl.BlockSpec((BM, 2000),
                     lambda i: (jnp.where(i < _H - 1, jnp.clip(i - _G, 0, 4),
                                          jnp.clip(i - _H, 0, 4)), 0)),
        full(w12_all),
    ]
    out_specs2 = [
        pl.BlockSpec((BM, C), lambda i: (jnp.clip(i - _F, 0, 9), 0)),
        pl.BlockSpec((BM, C), lambda i: (jnp.clip(i - _H, 0, 4), 0)),
    ]
    out_shape2 = [jax.ShapeDtypeStruct((4000, C), F32),
                  jax.ShapeDtypeStruct((2000, C), F32)]
    scratch2 = [pltpu.VMEM((4000, 3 * C), BF16),
                pltpu.VMEM((4000, 3 * C), BF16),
                pltpu.VMEM((2000, 2 * C), BF16),
                pltpu.VMEM((2000, 2 * C), BF16)]

    y1, y2 = pl.pallas_call(
        _call2_body,
        grid=(_N2,),
        in_specs=in_specs2,
        out_specs=out_specs2,
        out_shape=out_shape2,
        scratch_shapes=scratch2,
    )(b1, b2, laplacian_down_1, laplacian_up_1,
      laplacian_down_2, laplacian_up_2, w12_all)
    return (y0, y1, y2)
